# Initial kernel scaffold; baseline (speedup 1.0000x reference)
import jax
import jax.numpy as jnp
from jax import lax
from jax.experimental import pallas as pl
from jax.experimental.pallas import tpu as pltpu
from jax.experimental.pallas import tpu_sc as plsc

N = 10000
E = 320000
IN_DIM = 128
OUT_DIM = 64
HEADS = 4

NC = 2
NS = 16
EPT = E // NS
K = 80
NB = EPT // K
ROWW = OUT_DIM + 16
RPT = 2 * N // NS
RCH = 125


def _tc_body(h_ref, w_ref, a1_ref, a2_ref, z_ref, s1_ref, s2_ref):
    hb = h_ref[...]
    zc = lax.dot_general(hb, w_ref[...], (((1,), (0,)), ((), ())),
                         preferred_element_type=jnp.float32)
    for hd in range(HEADS):
        zh = zc[:, hd * OUT_DIM:(hd + 1) * OUT_DIM]
        z_ref[hd] = zh
        s1_ref[hd] = lax.dot_general(zh, a1_ref[...], (((1,), (0,)), ((), ())),
                                     preferred_element_type=jnp.float32)
        s2_ref[hd] = lax.dot_general(zh, a2_ref[...], (((1,), (0,)), ((), ())),
                                     preferred_element_type=jnp.float32)


_BN = 1000


def _tc_call(h, wcat, a1, a2):
    return pl.pallas_call(
        _tc_body,
        grid=(N // _BN,),
        in_specs=[
            pl.BlockSpec((_BN, IN_DIM), lambda i: (i, 0)),
            pl.BlockSpec((IN_DIM, HEADS * OUT_DIM), lambda i: (0, 0)),
            pl.BlockSpec((OUT_DIM, 1), lambda i: (0, 0)),
            pl.BlockSpec((OUT_DIM, 1), lambda i: (0, 0)),
        ],
        out_specs=[
            pl.BlockSpec((HEADS, _BN, OUT_DIM), lambda i: (0, i, 0)),
            pl.BlockSpec((HEADS, _BN, 1), lambda i: (0, i, 0)),
            pl.BlockSpec((HEADS, _BN, 1), lambda i: (0, i, 0)),
        ],
        out_shape=[
            jax.ShapeDtypeStruct((HEADS, N, OUT_DIM), jnp.float32),
            jax.ShapeDtypeStruct((HEADS, N, 1), jnp.float32),
            jax.ShapeDtypeStruct((HEADS, N, 1), jnp.float32),
        ],
    )(h, wcat, a1, a2)


def _sc_body(z_hbm, ssrc_hbm, sdst_hbm, src_hbm, dst_hbm, out_hbm,
             acc, src_v, dst_v, ssrc_v, sdst_v, zg, rowbuf, wbuf,
             zidx, didx, ebuf, obuf, sem0, sem1):
    c = lax.axis_index("c")
    s = lax.axis_index("s")
    zero16 = jnp.zeros((16,), jnp.float32)

    e0 = s * EPT
    pltpu.sync_copy(src_hbm.at[pl.ds(e0, EPT)], src_v)
    pltpu.sync_copy(dst_hbm.at[pl.ds(e0, EPT)], dst_v)
    for hl in range(2):
        head = 2 * c + hl
        pltpu.sync_copy(ssrc_hbm.at[head], ssrc_v.at[hl])
        pltpu.sync_copy(sdst_hbm.at[head], sdst_v.at[hl])

    def _zr(r, _):
        for j in range(ROWW // 16):
            ebuf[r, pl.ds(16 * j, 16)] = zero16
        return 0
    lax.fori_loop(0, RCH, _zr, 0)
    row0 = s * RPT
    for kk in range(RPT // RCH):
        pltpu.sync_copy(ebuf, acc.at[pl.ds(row0 + kk * RCH, RCH)])
    plsc.subcore_barrier()

    def _mk_idx(g, hl, p):
        base = g * K
        for grp in range(K // 16):
            off = base + grp * 16
            sv = src_v[pl.ds(off, 16)]
            dv = dst_v[pl.ds(off, 16)]
            s1 = plsc.load_gather(ssrc_v.at[hl], [sv])
            s2 = plsc.load_gather(sdst_v.at[hl], [dv])
            e = s1 + s2
            e = jnp.where(e > 0.0, e, e * jnp.float32(0.01))
            wbuf[p, pl.ds(grp * 16, 16)] = jnp.exp(e)
            zidx[p, 0, pl.ds(grp * 16, 16)] = sv + (2 * c + hl) * N
            didx[p, 0, pl.ds(grp * 16, 16)] = dv + hl * N

    def _gather_start(p, sem):
        pltpu.async_copy(z_hbm.at[zidx.at[p, 0]], zg.at[p], sem)

    def _mul(p):
        def mb(r, _):
            w_s = wbuf[p, r]
            for j in range(OUT_DIM // 16):
                rowbuf[r, pl.ds(16 * j, 16)] = zg[p, r, pl.ds(16 * j, 16)] * w_s
            rowbuf[r, pl.ds(OUT_DIM, 16)] = jnp.full((16,), w_s, jnp.float32)
            return 0
        lax.fori_loop(0, K, mb, 0)

    sems = (sem0, sem1)
    for hl in range(2):
        _mk_idx(0, hl, 0)
        _gather_start(0, sems[0])

        def _lb(t, _, hl=hl):
            for p in (0, 1):
                g = 2 * t + p
                nxt = 1 - p

                def _pref(hl=hl, g=g, nxt=nxt):
                    _mk_idx(g + 1, hl, nxt)
                    _gather_start(nxt, sems[nxt])
                pl.when(g + 1 < NB)(_pref)
                pltpu.make_async_copy(z_hbm.at[zidx.at[p, 0]], zg.at[p],
                                      sems[p]).wait()
                _mul(p)
                pltpu.sync_copy(rowbuf, acc.at[didx.at[p, 0]], add=True)
            return 0
        lax.fori_loop(0, NB // 2, _lb, 0)

    plsc.subcore_barrier()

    hl_e = s // 8
    n0 = row0 - hl_e * N
    col = (2 * c + hl_e) * OUT_DIM
    for kk in range(RPT // RCH):
        pltpu.sync_copy(acc.at[pl.ds(row0 + kk * RCH, RCH)], ebuf)

        def _db(r, _):
            den = ebuf[r, pl.ds(OUT_DIM, 16)]
            inv = jnp.where(den > 0.0, 1.0 / den, 0.0)
            for j in range(OUT_DIM // 16):
                obuf[r, pl.ds(16 * j, 16)] = ebuf[r, pl.ds(16 * j, 16)] * inv
            return 0
        lax.fori_loop(0, RCH, _db, 0)
        pltpu.sync_copy(obuf,
                        out_hbm.at[pl.ds(n0 + kk * RCH, RCH),
                                   pl.ds(col, OUT_DIM)])


def _sc_call(z_flat, ssrc, sdst, src, dst):
    mesh = plsc.VectorSubcoreMesh(core_axis_name="c", subcore_axis_name="s",
                                  num_cores=NC, num_subcores=NS)
    f = pl.kernel(
        _sc_body,
        out_type=jax.ShapeDtypeStruct((N, HEADS * OUT_DIM), jnp.float32),
        mesh=mesh,
        scratch_types=[
            pltpu.VMEM_SHARED((2 * N, ROWW), jnp.float32),
            pltpu.VMEM((EPT,), jnp.int32),
            pltpu.VMEM((EPT,), jnp.int32),
            pltpu.VMEM((2, N), jnp.float32),
            pltpu.VMEM((2, N), jnp.float32),
            pltpu.VMEM((2, K, OUT_DIM), jnp.float32),
            pltpu.VMEM((K, ROWW), jnp.float32),
            pltpu.VMEM((2, K), jnp.float32),
            pltpu.VMEM((2, 1, K), jnp.int32),
            pltpu.VMEM((2, 1, K), jnp.int32),
            pltpu.VMEM((RCH, ROWW), jnp.float32),
            pltpu.VMEM((RCH, OUT_DIM), jnp.float32),
            pltpu.SemaphoreType.DMA,
            pltpu.SemaphoreType.DMA,
        ],
    )
    return f(z_flat, ssrc, sdst, src, dst)


def kernel(h, edge_index, W, a):
    wcat = W.transpose(2, 0, 1).reshape(IN_DIM, HEADS * OUT_DIM)
    a1 = a[0, 0, :OUT_DIM].reshape(OUT_DIM, 1)
    a2 = a[0, 0, OUT_DIM:].reshape(OUT_DIM, 1)
    z4, s1, s2 = _tc_call(h, wcat, a1, a2)
    z_flat = z4.reshape(HEADS * N, OUT_DIM)
    ssrc = s1.reshape(HEADS, N)
    sdst = s2.reshape(HEADS, N)
    return _sc_call(z_flat, ssrc, sdst, edge_index[0], edge_index[1])


# speedup vs baseline: 21.5938x; 21.5938x over previous
import jax
import jax.numpy as jnp
from jax import lax
from jax.experimental import pallas as pl
from jax.experimental.pallas import tpu as pltpu
from jax.experimental.pallas import tpu_sc as plsc

N = 10000
E = 320000
IN_DIM = 128
OUT_DIM = 64
HEADS = 4

NC = 2
NS = 16
EPT = E // NS
K = 80
NB = EPT // K
ROWW = OUT_DIM + 16
RPT = N // NS
RCH = 125


def _tc_body(h_ref, w_ref, a1_ref, a2_ref, z_ref, s1_ref, s2_ref):
    hb = h_ref[...]
    zc = lax.dot_general(hb, w_ref[...], (((1,), (0,)), ((), ())),
                         preferred_element_type=jnp.float32)
    for hd in range(HEADS):
        zh = zc[:, hd * OUT_DIM:(hd + 1) * OUT_DIM]
        z_ref[hd] = zh
        s1_ref[hd] = lax.dot_general(zh, a1_ref[...], (((1,), (0,)), ((), ())),
                                     preferred_element_type=jnp.float32)
        s2_ref[hd] = lax.dot_general(zh, a2_ref[...], (((1,), (0,)), ((), ())),
                                     preferred_element_type=jnp.float32)


_BN = 1000


def _tc_call(h, wcat, a1, a2):
    return pl.pallas_call(
        _tc_body,
        grid=(N // _BN,),
        in_specs=[
            pl.BlockSpec((_BN, IN_DIM), lambda i: (i, 0)),
            pl.BlockSpec((IN_DIM, HEADS * OUT_DIM), lambda i: (0, 0)),
            pl.BlockSpec((OUT_DIM, 1), lambda i: (0, 0)),
            pl.BlockSpec((OUT_DIM, 1), lambda i: (0, 0)),
        ],
        out_specs=[
            pl.BlockSpec((HEADS, _BN, OUT_DIM), lambda i: (0, i, 0)),
            pl.BlockSpec((HEADS, _BN, 1), lambda i: (0, i, 0)),
            pl.BlockSpec((HEADS, _BN, 1), lambda i: (0, i, 0)),
        ],
        out_shape=[
            jax.ShapeDtypeStruct((HEADS, N, OUT_DIM), jnp.float32),
            jax.ShapeDtypeStruct((HEADS, N, 1), jnp.float32),
            jax.ShapeDtypeStruct((HEADS, N, 1), jnp.float32),
        ],
    )(h, wcat, a1, a2)


def _sc_body(z_hbm, ssrc_hbm, sdst_hbm, src_hbm, dst_hbm, out_hbm,
             acc, ssrc_v, sdst_v, srcb, dstb, wbuf, zidx, didx, zg, rowbuf,
             ebuf, obuf, sem_e0, sem_e1, sem_g0, sem_g1):
    c = lax.axis_index("c")
    s = lax.axis_index("s")
    zero16 = jnp.zeros((16,), jnp.float32)
    e0 = s * EPT
    sem_e = (sem_e0, sem_e1)
    sem_g = (sem_g0, sem_g1)
    row0 = s * RPT

    def stage_edges(g, p):
        off = e0 + g * K
        pltpu.async_copy(src_hbm.at[pl.ds(off, K)], srcb.at[p], sem_e[p])
        pltpu.async_copy(dst_hbm.at[pl.ds(off, K)], dstb.at[p], sem_e[p])

    def wait_edges(g, p):
        off = e0 + g * K
        pltpu.make_async_copy(src_hbm.at[pl.ds(off, K)], srcb.at[p],
                              sem_e[p]).wait()
        pltpu.make_async_copy(dst_hbm.at[pl.ds(off, K)], dstb.at[p],
                              sem_e[p]).wait()

    for hl in range(2):
        head = 2 * c + hl
        pltpu.sync_copy(ssrc_hbm.at[head], ssrc_v)
        pltpu.sync_copy(sdst_hbm.at[head], sdst_v)

        def _zr(r, _):
            for j in range(ROWW // 16):
                ebuf[r, pl.ds(16 * j, 16)] = zero16
            return 0
        lax.fori_loop(0, RCH, _zr, 0)
        for kk in range(RPT // RCH):
            pltpu.sync_copy(ebuf, acc.at[pl.ds(row0 + kk * RCH, RCH)])
        plsc.subcore_barrier()

        def _mk_idx(p, head=head):
            for grp in range(K // 16):
                sv = srcb[p, pl.ds(grp * 16, 16)]
                dv = dstb[p, pl.ds(grp * 16, 16)]
                s1 = plsc.load_gather(ssrc_v, [sv])
                s2 = plsc.load_gather(sdst_v, [dv])
                e = s1 + s2
                e = jnp.where(e > 0.0, e, e * jnp.float32(0.01))
                wbuf[p, pl.ds(grp * 16, 16)] = jnp.exp(e)
                zidx[p, 0, pl.ds(grp * 16, 16)] = sv + head * N
                didx[p, 0, pl.ds(grp * 16, 16)] = dv

        def _gather(p):
            pltpu.async_copy(z_hbm.at[zidx.at[p, 0]], zg.at[p], sem_g[p])

        def _wait_gather(p):
            pltpu.make_async_copy(z_hbm.at[zidx.at[p, 0]], zg.at[p],
                                  sem_g[p]).wait()

        def _mul_scatter(p):
            def mb(q, _):
                wv = wbuf[p, pl.ds(16 * q, 16)]
                for e_i in range(16):
                    r = 16 * q + e_i
                    w_s = wv[e_i]
                    for j in range(OUT_DIM // 16):
                        rowbuf[r, pl.ds(16 * j, 16)] = (
                            zg[p, r, pl.ds(16 * j, 16)] * w_s)
                    rowbuf[r, pl.ds(OUT_DIM, 16)] = jnp.full((16,), w_s,
                                                             jnp.float32)
                return 0
            lax.fori_loop(0, K // 16, mb, 0)
            pltpu.sync_copy(rowbuf, acc.at[didx.at[p, 0]], add=True)

        stage_edges(0, 0)
        stage_edges(1, 1)
        wait_edges(0, 0)
        _mk_idx(0)
        _gather(0)

        def _lb(t, _):
            for p in (0, 1):
                g = 2 * t + p
                pl.when(g + 2 < NB)(lambda g=g, p=p: stage_edges(g + 2, p))

                def _x(g=g, p=p):
                    wait_edges(g + 1, 1 - p)
                    _mk_idx(1 - p)
                    _gather(1 - p)
                pl.when(g + 1 < NB)(_x)
                _wait_gather(p)
                _mul_scatter(p)
            return 0
        lax.fori_loop(0, NB // 2, _lb, 0)

        plsc.subcore_barrier()

        col = head * OUT_DIM
        for kk in range(RPT // RCH):
            pltpu.sync_copy(acc.at[pl.ds(row0 + kk * RCH, RCH)], ebuf)

            def _db(r, _):
                den = ebuf[r, pl.ds(OUT_DIM, 16)]
                inv = jnp.where(den > 0.0, 1.0 / den, 0.0)
                for j in range(OUT_DIM // 16):
                    obuf[r, pl.ds(16 * j, 16)] = (
                        ebuf[r, pl.ds(16 * j, 16)] * inv)
                return 0
            lax.fori_loop(0, RCH, _db, 0)
            pltpu.sync_copy(obuf,
                            out_hbm.at[pl.ds(row0 + kk * RCH, RCH),
                                       pl.ds(col, OUT_DIM)])
        plsc.subcore_barrier()


def _sc_call(z_flat, ssrc, sdst, src, dst):
    mesh = plsc.VectorSubcoreMesh(core_axis_name="c", subcore_axis_name="s",
                                  num_cores=NC, num_subcores=NS)
    f = pl.kernel(
        _sc_body,
        out_type=jax.ShapeDtypeStruct((N, HEADS * OUT_DIM), jnp.float32),
        mesh=mesh,
        compiler_params=pltpu.CompilerParams(use_tc_tiling_on_sc=False,
                                             needs_layout_passes=False),
        scratch_types=[
            pltpu.VMEM_SHARED((N, ROWW), jnp.float32),
            pltpu.VMEM((N,), jnp.float32),
            pltpu.VMEM((N,), jnp.float32),
            pltpu.VMEM((2, K), jnp.int32),
            pltpu.VMEM((2, K), jnp.int32),
            pltpu.VMEM((2, K), jnp.float32),
            pltpu.VMEM((2, 1, K), jnp.int32),
            pltpu.VMEM((2, 1, K), jnp.int32),
            pltpu.VMEM((2, K, OUT_DIM), jnp.float32),
            pltpu.VMEM((K, ROWW), jnp.float32),
            pltpu.VMEM((RCH, ROWW), jnp.float32),
            pltpu.VMEM((RCH, OUT_DIM), jnp.float32),
            pltpu.SemaphoreType.DMA,
            pltpu.SemaphoreType.DMA,
            pltpu.SemaphoreType.DMA,
            pltpu.SemaphoreType.DMA,
        ],
    )
    return f(z_flat, ssrc, sdst, src, dst)


def kernel(h, edge_index, W, a):
    wcat = W.transpose(2, 0, 1).reshape(IN_DIM, HEADS * OUT_DIM)
    a1 = a[0, 0, :OUT_DIM].reshape(OUT_DIM, 1)
    a2 = a[0, 0, OUT_DIM:].reshape(OUT_DIM, 1)
    z4, s1, s2 = _tc_call(h, wcat, a1, a2)
    z_flat = z4.reshape(HEADS * N, OUT_DIM)
    ssrc = s1.reshape(HEADS, N)
    sdst = s2.reshape(HEADS, N)
    return _sc_call(z_flat, ssrc, sdst, edge_index[0], edge_index[1])


# speedup vs baseline: 25.0930x; 1.1620x over previous
import jax
import jax.numpy as jnp
from jax import lax
from jax.experimental import pallas as pl
from jax.experimental.pallas import tpu as pltpu
from jax.experimental.pallas import tpu_sc as plsc

N = 10000
E = 320000
IN_DIM = 128
OUT_DIM = 64
HEADS = 4

NC = 2
NS = 16
EPT = E // NS
K = 80
NB = EPT // K
ROWW = OUT_DIM + 16
RPT = N // NS
RCH = 25


def _tc_body(h_ref, w_ref, a1_ref, a2_ref, z_ref, s1_ref, s2_ref):
    hb = h_ref[...]
    zc = lax.dot_general(hb, w_ref[...], (((1,), (0,)), ((), ())),
                         preferred_element_type=jnp.float32)
    for hd in range(HEADS):
        zh = zc[:, hd * OUT_DIM:(hd + 1) * OUT_DIM]
        z_ref[hd] = zh
        s1_ref[hd] = lax.dot_general(zh, a1_ref[...], (((1,), (0,)), ((), ())),
                                     preferred_element_type=jnp.float32)
        s2_ref[hd] = lax.dot_general(zh, a2_ref[...], (((1,), (0,)), ((), ())),
                                     preferred_element_type=jnp.float32)


_BN = 1000


def _tc_call(h, wcat, a1, a2):
    return pl.pallas_call(
        _tc_body,
        grid=(N // _BN,),
        in_specs=[
            pl.BlockSpec((_BN, IN_DIM), lambda i: (i, 0)),
            pl.BlockSpec((IN_DIM, HEADS * OUT_DIM), lambda i: (0, 0)),
            pl.BlockSpec((OUT_DIM, 1), lambda i: (0, 0)),
            pl.BlockSpec((OUT_DIM, 1), lambda i: (0, 0)),
        ],
        out_specs=[
            pl.BlockSpec((HEADS, _BN, OUT_DIM), lambda i: (0, i, 0)),
            pl.BlockSpec((HEADS, _BN, 1), lambda i: (0, i, 0)),
            pl.BlockSpec((HEADS, _BN, 1), lambda i: (0, i, 0)),
        ],
        out_shape=[
            jax.ShapeDtypeStruct((HEADS, N, OUT_DIM), jnp.float32),
            jax.ShapeDtypeStruct((HEADS, N, 1), jnp.float32),
            jax.ShapeDtypeStruct((HEADS, N, 1), jnp.float32),
        ],
    )(h, wcat, a1, a2)


def _sc_body(z_hbm, ssrc_hbm, sdst_hbm, src_hbm, dst_hbm, out_hbm,
             acc, ssrc_v, sdst_v, srcb, dstb, wbuf, zidx, didx, zg,
             rowbuf, ebuf, sem_e0, sem_e1, sem_g0, sem_g1, sem_s0, sem_s1):
    c = lax.axis_index("c")
    s = lax.axis_index("s")
    zero16 = jnp.zeros((16,), jnp.float32)
    e0 = s * EPT
    sem_e = (sem_e0, sem_e1)
    sem_g = (sem_g0, sem_g1)
    sem_s = (sem_s0, sem_s1)
    row0 = s * RPT

    def stage_edges(g, p):
        off = e0 + g * K
        pltpu.async_copy(src_hbm.at[pl.ds(off, K)], srcb.at[p], sem_e[p])
        pltpu.async_copy(dst_hbm.at[pl.ds(off, K)], dstb.at[p], sem_e[p])

    def wait_edges(g, p):
        off = e0 + g * K
        pltpu.make_async_copy(src_hbm.at[pl.ds(off, K)], srcb.at[p],
                              sem_e[p]).wait()
        pltpu.make_async_copy(dst_hbm.at[pl.ds(off, K)], dstb.at[p],
                              sem_e[p]).wait()

    for hl in range(2):
        head = 2 * c + hl
        pltpu.sync_copy(ssrc_hbm.at[head], ssrc_v)
        pltpu.sync_copy(sdst_hbm.at[head], sdst_v)

        def _zr(r, _):
            for j in range(ROWW // 16):
                ebuf[r, pl.ds(16 * j, 16)] = zero16
            return 0
        lax.fori_loop(0, RCH, _zr, 0)
        for kk in range(RPT // RCH):
            pltpu.sync_copy(ebuf, acc.at[pl.ds(row0 + kk * RCH, RCH)])
        plsc.subcore_barrier()

        def _mk_idx(p, head=head):
            for grp in range(K // 16):
                sv = srcb[p, pl.ds(grp * 16, 16)]
                dv = dstb[p, pl.ds(grp * 16, 16)]
                s1 = plsc.load_gather(ssrc_v, [sv])
                s2 = plsc.load_gather(sdst_v, [dv])
                e = s1 + s2
                e = jnp.where(e > 0.0, e, e * jnp.float32(0.01))
                wbuf[p, pl.ds(grp * 16, 16)] = jnp.exp(e)
                zidx[p, 0, pl.ds(grp * 16, 16)] = sv + head * N
                didx[p, 0, pl.ds(grp * 16, 16)] = dv

        def _gather(p):
            pltpu.async_copy(z_hbm.at[zidx.at[p, 0]], zg.at[p], sem_g[p])

        def _wait_gather(p):
            pltpu.make_async_copy(z_hbm.at[zidx.at[p, 0]], zg.at[p],
                                  sem_g[p]).wait()

        def _wait_scatter(p):
            pltpu.make_async_copy(rowbuf.at[p], acc.at[didx.at[p, 0]],
                                  sem_s[p]).wait()

        def _mul_scatter(p):
            def mb(q, _):
                wv = wbuf[p, pl.ds(16 * q, 16)]
                for e_i in range(16):
                    r = 16 * q + e_i
                    w_s = wv[e_i]
                    for j in range(OUT_DIM // 16):
                        rowbuf[p, r, pl.ds(16 * j, 16)] = (
                            zg[p, r, pl.ds(16 * j, 16)] * w_s)
                    rowbuf[p, r, pl.ds(OUT_DIM, 16)] = jnp.full((16,), w_s,
                                                                jnp.float32)
                return 0
            lax.fori_loop(0, K // 16, mb, 0)
            pltpu.async_copy(rowbuf.at[p], acc.at[didx.at[p, 0]], sem_s[p],
                             add=True)

        stage_edges(0, 0)
        stage_edges(1, 1)
        wait_edges(0, 0)
        _mk_idx(0)
        _gather(0)

        def _lb(t, _):
            for p in (0, 1):
                g = 2 * t + p
                pl.when(g + 2 < NB)(lambda g=g, p=p: stage_edges(g + 2, p))

                def _x(g=g, p=p):
                    wait_edges(g + 1, 1 - p)
                    _mk_idx(1 - p)
                    _gather(1 - p)
                pl.when(g + 1 < NB)(_x)
                _wait_gather(p)
                pl.when(t > 0)(lambda p=p: _wait_scatter(p))
                _mul_scatter(p)
            return 0
        lax.fori_loop(0, NB // 2, _lb, 0)
        _wait_scatter(0)
        _wait_scatter(1)

        plsc.subcore_barrier()

        col = head * OUT_DIM
        for kk in range(RPT // RCH):
            pltpu.sync_copy(acc.at[pl.ds(row0 + kk * RCH, RCH)], ebuf)

            def _db(r, _):
                den = ebuf[r, pl.ds(OUT_DIM, 16)]
                inv = jnp.where(den > 0.0, 1.0 / den, 0.0)
                for j in range(OUT_DIM // 16):
                    ebuf[r, pl.ds(16 * j, 16)] = (
                        ebuf[r, pl.ds(16 * j, 16)] * inv)
                return 0
            lax.fori_loop(0, RCH, _db, 0)
            pltpu.sync_copy(ebuf.at[:, pl.ds(0, OUT_DIM)],
                            out_hbm.at[pl.ds(row0 + kk * RCH, RCH),
                                       pl.ds(col, OUT_DIM)])
        plsc.subcore_barrier()


def _sc_call(z_flat, ssrc, sdst, src, dst):
    mesh = plsc.VectorSubcoreMesh(core_axis_name="c", subcore_axis_name="s",
                                  num_cores=NC, num_subcores=NS)
    f = pl.kernel(
        _sc_body,
        out_type=jax.ShapeDtypeStruct((N, HEADS * OUT_DIM), jnp.float32),
        mesh=mesh,
        compiler_params=pltpu.CompilerParams(use_tc_tiling_on_sc=False,
                                             needs_layout_passes=False),
        scratch_types=[
            pltpu.VMEM_SHARED((N, ROWW), jnp.float32),
            pltpu.VMEM((N,), jnp.float32),
            pltpu.VMEM((N,), jnp.float32),
            pltpu.VMEM((2, K), jnp.int32),
            pltpu.VMEM((2, K), jnp.int32),
            pltpu.VMEM((2, K), jnp.float32),
            pltpu.VMEM((2, 1, K), jnp.int32),
            pltpu.VMEM((2, 1, K), jnp.int32),
            pltpu.VMEM((2, K, OUT_DIM), jnp.float32),
            pltpu.VMEM((2, K, ROWW), jnp.float32),
            pltpu.VMEM((RCH, ROWW), jnp.float32),
            pltpu.SemaphoreType.DMA,
            pltpu.SemaphoreType.DMA,
            pltpu.SemaphoreType.DMA,
            pltpu.SemaphoreType.DMA,
            pltpu.SemaphoreType.DMA,
            pltpu.SemaphoreType.DMA,
        ],
    )
    return f(z_flat, ssrc, sdst, src, dst)


def kernel(h, edge_index, W, a):
    wcat = W.transpose(2, 0, 1).reshape(IN_DIM, HEADS * OUT_DIM)
    a1 = a[0, 0, :OUT_DIM].reshape(OUT_DIM, 1)
    a2 = a[0, 0, OUT_DIM:].reshape(OUT_DIM, 1)
    z4, s1, s2 = _tc_call(h, wcat, a1, a2)
    z_flat = z4.reshape(HEADS * N, OUT_DIM)
    ssrc = s1.reshape(HEADS, N)
    sdst = s2.reshape(HEADS, N)
    return _sc_call(z_flat, ssrc, sdst, edge_index[0], edge_index[1])


# speedup vs baseline: 48.1581x; 1.9192x over previous
import jax
import jax.numpy as jnp
from jax import lax
from jax.experimental import pallas as pl
from jax.experimental.pallas import tpu as pltpu
from jax.experimental.pallas import tpu_sc as plsc

N = 10000
E = 320000
IN_DIM = 128
OUT_DIM = 64
HEADS = 4

NC = 2
NS = 16
EPT = E // NS
K = 80
NB = EPT // K
ROWW = OUT_DIM + 16
RPT = N // NS
RCH = 25


def _tc_body(h_ref, w_ref, a1_ref, a2_ref, z_ref, s1_ref, s2_ref):
    hb = h_ref[...]
    zc = lax.dot_general(hb, w_ref[...], (((1,), (0,)), ((), ())),
                         preferred_element_type=jnp.float32)
    for hd in range(HEADS):
        zh = zc[:, hd * OUT_DIM:(hd + 1) * OUT_DIM]
        z_ref[hd] = zh
        s1_ref[hd] = lax.dot_general(zh, a1_ref[...], (((1,), (0,)), ((), ())),
                                     preferred_element_type=jnp.float32)
        s2_ref[hd] = lax.dot_general(zh, a2_ref[...], (((1,), (0,)), ((), ())),
                                     preferred_element_type=jnp.float32)


_BN = 1000


def _tc_call(h, wcat, a1, a2):
    return pl.pallas_call(
        _tc_body,
        grid=(N // _BN,),
        in_specs=[
            pl.BlockSpec((_BN, IN_DIM), lambda i: (i, 0)),
            pl.BlockSpec((IN_DIM, HEADS * OUT_DIM), lambda i: (0, 0)),
            pl.BlockSpec((OUT_DIM, 1), lambda i: (0, 0)),
            pl.BlockSpec((OUT_DIM, 1), lambda i: (0, 0)),
        ],
        out_specs=[
            pl.BlockSpec((HEADS, _BN, OUT_DIM), lambda i: (0, i, 0)),
            pl.BlockSpec((HEADS, _BN, 1), lambda i: (0, i, 0)),
            pl.BlockSpec((HEADS, _BN, 1), lambda i: (0, i, 0)),
        ],
        out_shape=[
            jax.ShapeDtypeStruct((HEADS, N, OUT_DIM), jnp.float32),
            jax.ShapeDtypeStruct((HEADS, N, 1), jnp.float32),
            jax.ShapeDtypeStruct((HEADS, N, 1), jnp.float32),
        ],
    )(h, wcat, a1, a2)


def _sc_body(z_hbm, ssrc_hbm, sdst_hbm, src_hbm, dst_hbm, out_hbm,
             acc, ssrc_v, sdst_v, srcb, dstb, wbuf, zidx, didx, sidx,
             zg, rowbuf, ebuf,
             sem_e0, sem_e1, sem_g0, sem_g1, sem_s0, sem_s1):
    c = lax.axis_index("c")
    s = lax.axis_index("s")
    zero16 = jnp.zeros((16,), jnp.float32)
    izero16 = jnp.zeros((16,), jnp.int32)
    ione16 = jnp.ones((16,), jnp.int32)
    e0 = s * EPT
    sem_e = (sem_e0, sem_e1)
    sem_g = (sem_g0, sem_g1)
    sem_s = (sem_s0, sem_s1)
    row0 = s * RPT

    def stage_edges(g, p):
        off = e0 + g * K
        pltpu.async_copy(src_hbm.at[pl.ds(off, K)], srcb.at[p], sem_e[p])
        pltpu.async_copy(dst_hbm.at[pl.ds(off, K)], dstb.at[p], sem_e[p])

    def wait_edges(g, p):
        off = e0 + g * K
        pltpu.make_async_copy(src_hbm.at[pl.ds(off, K)], srcb.at[p],
                              sem_e[p]).wait()
        pltpu.make_async_copy(dst_hbm.at[pl.ds(off, K)], dstb.at[p],
                              sem_e[p]).wait()

    for hl in range(2):
        head = 2 * c + hl
        pltpu.sync_copy(ssrc_hbm.at[head], ssrc_v)
        pltpu.sync_copy(sdst_hbm.at[head], sdst_v)

        def _zr(r, _):
            for j in range(ROWW // 16):
                ebuf[r, pl.ds(16 * j, 16)] = zero16
            return 0
        lax.fori_loop(0, RCH, _zr, 0)
        for kk in range(RPT // RCH):
            pltpu.sync_copy(ebuf, acc.at[pl.ds(row0 + kk * RCH, RCH)])
        plsc.subcore_barrier()

        def _mk_w(p, head=head):
            for grp in range(K // 16):
                sv = srcb[p, pl.ds(grp * 16, 16)]
                dv = dstb[p, pl.ds(grp * 16, 16)]
                s1 = plsc.load_gather(ssrc_v, [sv])
                s2 = plsc.load_gather(sdst_v, [dv])
                e = s1 + s2
                e = jnp.where(e > 0.0, e, e * jnp.float32(0.01))
                wbuf[p, pl.ds(grp * 16, 16)] = jnp.exp(e)
                zidx[p, 0, pl.ds(grp * 16, 16)] = sv + head * N
                didx[p, 0, pl.ds(grp * 16, 16)] = dv
            pltpu.async_copy(z_hbm.at[zidx.at[p, 0]], zg.at[p], sem_g[p])

        def _wait_gather(p):
            pltpu.make_async_copy(z_hbm.at[zidx.at[p, 0]], zg.at[p],
                                  sem_g[p]).wait()

        def _wait_scatter(p):
            pltpu.make_async_copy(rowbuf.at[p], acc.at[sidx.at[p, 0]],
                                  sem_s[p]).wait()

        def _mul_scatter(p):
            def mb(q, _):
                wv = wbuf[p, pl.ds(16 * q, 16)]
                for e2 in range(4):
                    blk = []
                    for e_i in range(4):
                        r = 16 * q + 4 * e2 + e_i
                        w_s = wv[4 * e2 + e_i]
                        vals = [zg[p, r, pl.ds(16 * j, 16)] * w_s
                                for j in range(OUT_DIM // 16)]
                        blk.append((r, w_s, vals))
                    for r, w_s, vals in blk:
                        for j in range(OUT_DIM // 16):
                            rowbuf[p, r, pl.ds(16 * j, 16)] = vals[j]
                        rowbuf[p, r, pl.ds(OUT_DIM, 16)] = jnp.full(
                            (16,), w_s, jnp.float32)
                return 0
            lax.fori_loop(0, K // 16, mb, 0)
            for grp in range(K // 16):
                sidx[p, 0, pl.ds(grp * 16, 16)] = (
                    didx[p, 0, pl.ds(grp * 16, 16)])
            pltpu.async_copy(rowbuf.at[p], acc.at[sidx.at[p, 0]], sem_s[p],
                             add=True)

        stage_edges(0, 0)
        stage_edges(1, 1)
        wait_edges(0, 0)
        _mk_w(0)

        def _lb(t, _):
            for p in (0, 1):
                g = 2 * t + p
                pl.when(g + 2 < NB)(lambda g=g, p=p: stage_edges(g + 2, p))

                def _x(g=g, p=p):
                    wait_edges(g + 1, 1 - p)
                    _mk_w(1 - p)
                pl.when(g + 1 < NB)(_x)
                _wait_gather(p)
                pl.when(t > 0)(lambda p=p: _wait_scatter(p))
                _mul_scatter(p)
            return 0
        lax.fori_loop(0, NB // 2, _lb, 0)
        _wait_scatter(0)
        _wait_scatter(1)

        plsc.subcore_barrier()

        col = head * OUT_DIM
        for kk in range(RPT // RCH):
            pltpu.sync_copy(acc.at[pl.ds(row0 + kk * RCH, RCH)], ebuf)

            def _db(r, _):
                den = ebuf[r, pl.ds(OUT_DIM, 16)]
                inv = jnp.where(den > 0.0, 1.0 / den, 0.0)
                for j in range(OUT_DIM // 16):
                    ebuf[r, pl.ds(16 * j, 16)] = (
                        ebuf[r, pl.ds(16 * j, 16)] * inv)
                return 0
            lax.fori_loop(0, RCH, _db, 0)
            pltpu.sync_copy(ebuf.at[:, pl.ds(0, OUT_DIM)],
                            out_hbm.at[pl.ds(row0 + kk * RCH, RCH),
                                       pl.ds(col, OUT_DIM)])
        plsc.subcore_barrier()


def _sc_call(z_flat, ssrc, sdst, src, dst):
    mesh = plsc.VectorSubcoreMesh(core_axis_name="c", subcore_axis_name="s",
                                  num_cores=NC, num_subcores=NS)
    f = pl.kernel(
        _sc_body,
        out_type=jax.ShapeDtypeStruct((N, HEADS * OUT_DIM), jnp.float32),
        mesh=mesh,
        compiler_params=pltpu.CompilerParams(use_tc_tiling_on_sc=False,
                                             needs_layout_passes=False),
        scratch_types=[
            pltpu.VMEM_SHARED((N, ROWW), jnp.float32),
            pltpu.VMEM((N,), jnp.float32),
            pltpu.VMEM((N,), jnp.float32),
            pltpu.VMEM((2, K), jnp.int32),
            pltpu.VMEM((2, K), jnp.int32),
            pltpu.VMEM((2, K), jnp.float32),
            pltpu.VMEM((2, 1, K), jnp.int32),
            pltpu.VMEM((2, 1, K), jnp.int32),
            pltpu.VMEM((2, 1, K), jnp.int32),
            pltpu.VMEM((2, K, OUT_DIM), jnp.float32),
            pltpu.VMEM((2, K, ROWW), jnp.float32),
            pltpu.VMEM((RCH, ROWW), jnp.float32),
            pltpu.SemaphoreType.DMA,
            pltpu.SemaphoreType.DMA,
            pltpu.SemaphoreType.DMA,
            pltpu.SemaphoreType.DMA,
            pltpu.SemaphoreType.DMA,
            pltpu.SemaphoreType.DMA,
        ],
    )
    return f(z_flat, ssrc, sdst, src, dst)


def kernel(h, edge_index, W, a):
    wcat = W.transpose(2, 0, 1).reshape(IN_DIM, HEADS * OUT_DIM)
    a1 = a[0, 0, :OUT_DIM].reshape(OUT_DIM, 1)
    a2 = a[0, 0, OUT_DIM:].reshape(OUT_DIM, 1)
    z4, s1, s2 = _tc_call(h, wcat, a1, a2)
    z_flat = z4.reshape(HEADS * N, OUT_DIM)
    ssrc = s1.reshape(HEADS, N)
    sdst = s2.reshape(HEADS, N)
    return _sc_call(z_flat, ssrc, sdst, edge_index[0], edge_index[1])


# speedup vs baseline: 57.4453x; 1.1928x over previous
import jax
import jax.numpy as jnp
from jax import lax
from jax.experimental import pallas as pl
from jax.experimental.pallas import tpu as pltpu
from jax.experimental.pallas import tpu_sc as plsc

N = 10000
E = 320000
IN_DIM = 128
OUT_DIM = 64
HEADS = 4

NC = 2
NS = 16
EPT = E // NS
K = 80
NB = EPT // K
ROWW = OUT_DIM + 16
RPT = N // NS
RCH = 25
_D = 5


def _tc_body(h_ref, w_ref, a1_ref, a2_ref, z_ref, s1_ref, s2_ref):
    hb = h_ref[...]
    zc = lax.dot_general(hb, w_ref[...], (((1,), (0,)), ((), ())),
                         preferred_element_type=jnp.float32)
    for hd in range(HEADS):
        zh = zc[:, hd * OUT_DIM:(hd + 1) * OUT_DIM]
        z_ref[hd] = zh
        s1_ref[hd] = lax.dot_general(zh, a1_ref[...], (((1,), (0,)), ((), ())),
                                     preferred_element_type=jnp.float32)
        s2_ref[hd] = lax.dot_general(zh, a2_ref[...], (((1,), (0,)), ((), ())),
                                     preferred_element_type=jnp.float32)


_BN = 1000


def _tc_call(h, wcat, a1, a2):
    return pl.pallas_call(
        _tc_body,
        grid=(N // _BN,),
        in_specs=[
            pl.BlockSpec((_BN, IN_DIM), lambda i: (i, 0)),
            pl.BlockSpec((IN_DIM, HEADS * OUT_DIM), lambda i: (0, 0)),
            pl.BlockSpec((OUT_DIM, 1), lambda i: (0, 0)),
            pl.BlockSpec((OUT_DIM, 1), lambda i: (0, 0)),
        ],
        out_specs=[
            pl.BlockSpec((HEADS, _BN, OUT_DIM), lambda i: (0, i, 0)),
            pl.BlockSpec((HEADS, _BN, 1), lambda i: (0, i, 0)),
            pl.BlockSpec((HEADS, _BN, 1), lambda i: (0, i, 0)),
        ],
        out_shape=[
            jax.ShapeDtypeStruct((HEADS, N, OUT_DIM), jnp.float32),
            jax.ShapeDtypeStruct((HEADS, N, 1), jnp.float32),
            jax.ShapeDtypeStruct((HEADS, N, 1), jnp.float32),
        ],
    )(h, wcat, a1, a2)


def _sc_body(z_hbm, ssrc_hbm, sdst_hbm, src_hbm, dst_hbm, out_hbm,
             acc, ssrc_v, sdst_v, srcb, dstb, wbuf, zidx, sidx,
             zg, rowbuf, *sems):
    c = lax.axis_index("c")
    s = lax.axis_index("s")
    zero16 = jnp.zeros((16,), jnp.float32)
    e0 = s * EPT
    sem_e = sems[0:_D]
    sem_g = sems[_D:2 * _D]
    sem_s = sems[2 * _D:3 * _D]
    row0 = s * RPT

    def stage_edges(g, p):
        off = e0 + g * K
        pltpu.async_copy(src_hbm.at[pl.ds(off, K)], srcb.at[p], sem_e[p])
        pltpu.async_copy(dst_hbm.at[pl.ds(off, K)], dstb.at[p], sem_e[p])

    def wait_edges(g, p):
        off = e0 + g * K
        pltpu.make_async_copy(src_hbm.at[pl.ds(off, K)], srcb.at[p],
                              sem_e[p]).wait()
        pltpu.make_async_copy(dst_hbm.at[pl.ds(off, K)], dstb.at[p],
                              sem_e[p]).wait()

    for hl in range(2):
        head = 2 * c + hl
        pltpu.sync_copy(ssrc_hbm.at[head], ssrc_v)
        pltpu.sync_copy(sdst_hbm.at[head], sdst_v)

        def _zr(r, _):
            for j in range(ROWW // 16):
                rowbuf[0, r, pl.ds(16 * j, 16)] = zero16
            return 0
        lax.fori_loop(0, RCH, _zr, 0)
        for kk in range(RPT // RCH):
            pltpu.sync_copy(rowbuf.at[0, pl.ds(0, RCH)],
                            acc.at[pl.ds(row0 + kk * RCH, RCH)])
        plsc.subcore_barrier()

        def _mk_w(p, head=head):
            for grp in range(K // 16):
                sv = srcb[p, pl.ds(grp * 16, 16)]
                dv = dstb[p, pl.ds(grp * 16, 16)]
                s1 = plsc.load_gather(ssrc_v, [sv])
                s2 = plsc.load_gather(sdst_v, [dv])
                e = s1 + s2
                e = jnp.where(e > 0.0, e, e * jnp.float32(0.01))
                wbuf[p, pl.ds(grp * 16, 16)] = jnp.exp(e)
                zidx[p, 0, pl.ds(grp * 16, 16)] = sv + head * N
            pltpu.async_copy(z_hbm.at[zidx.at[p, 0]], zg.at[p], sem_g[p])

        def _wait_gather(p):
            pltpu.make_async_copy(z_hbm.at[zidx.at[p, 0]], zg.at[p],
                                  sem_g[p]).wait()

        def _wait_scatter(p):
            pltpu.make_async_copy(rowbuf.at[p], acc.at[sidx.at[p, 0]],
                                  sem_s[p]).wait()

        def _mul_scatter(p, b=None):
            def mb(q, _):
                wv = wbuf[p, pl.ds(16 * q, 16)]
                for e2 in range(4):
                    blk = []
                    for e_i in range(4):
                        r = 16 * q + 4 * e2 + e_i
                        w_s = wv[4 * e2 + e_i]
                        vals = [zg[p, r, pl.ds(16 * j, 16)] * w_s
                                for j in range(OUT_DIM // 16)]
                        blk.append((r, w_s, vals))
                    for r, w_s, vals in blk:
                        for j in range(OUT_DIM // 16):
                            rowbuf[p, r, pl.ds(16 * j, 16)] = vals[j]
                        rowbuf[p, r, pl.ds(OUT_DIM, 16)] = jnp.full(
                            (16,), w_s, jnp.float32)
                return 0
            lax.fori_loop(0, K // 16, mb, 0)
            for grp in range(K // 16):
                sidx[p, 0, pl.ds(grp * 16, 16)] = (
                    dstb[p, pl.ds(grp * 16, 16)])
            if b is not None:
                pl.when(b > 0)(
                    lambda: _wait_scatter((p + _D - 1) % _D))
            pltpu.async_copy(rowbuf.at[p], acc.at[sidx.at[p, 0]], sem_s[p],
                             add=True)

        for g in range(_D - 1):
            stage_edges(g, g)
        for g in range(2):
            wait_edges(g, g)
            _mk_w(g)

        def _lb(t, _):
            for i in range(_D):
                b = _D * t + i
                sS = (i + 4) % _D
                sX = (i + 2) % _D
                pl.when(b + 4 < NB)(
                    lambda b=b, sS=sS: stage_edges(b + 4, sS))

                def _x(b=b, sX=sX):
                    wait_edges(b + 2, sX)
                    _mk_w(sX)
                pl.when(b + 2 < NB)(_x)
                _wait_gather(i)
                _mul_scatter(i, b)
            return 0
        lax.fori_loop(0, NB // _D, _lb, 0)
        _wait_scatter((NB - 1) % _D)

        plsc.subcore_barrier()

        col = head * OUT_DIM
        for kk in range(RPT // RCH):
            pltpu.sync_copy(acc.at[pl.ds(row0 + kk * RCH, RCH)],
                            rowbuf.at[0, pl.ds(0, RCH)])

            def _db(r, _):
                den = rowbuf[0, r, pl.ds(OUT_DIM, 16)]
                inv = jnp.where(den > 0.0, 1.0 / den, 0.0)
                for j in range(OUT_DIM // 16):
                    rowbuf[0, r, pl.ds(16 * j, 16)] = (
                        rowbuf[0, r, pl.ds(16 * j, 16)] * inv)
                return 0
            lax.fori_loop(0, RCH, _db, 0)
            pltpu.sync_copy(rowbuf.at[0, pl.ds(0, RCH), pl.ds(0, OUT_DIM)],
                            out_hbm.at[pl.ds(row0 + kk * RCH, RCH),
                                       pl.ds(col, OUT_DIM)])
        plsc.subcore_barrier()


def _sc_call(z_flat, ssrc, sdst, src, dst):
    mesh = plsc.VectorSubcoreMesh(core_axis_name="c", subcore_axis_name="s",
                                  num_cores=NC, num_subcores=NS)
    f = pl.kernel(
        _sc_body,
        out_type=jax.ShapeDtypeStruct((N, HEADS * OUT_DIM), jnp.float32),
        mesh=mesh,
        compiler_params=pltpu.CompilerParams(use_tc_tiling_on_sc=False,
                                             needs_layout_passes=False),
        scratch_types=[
            pltpu.VMEM_SHARED((N, ROWW), jnp.float32),
            pltpu.VMEM((N,), jnp.float32),
            pltpu.VMEM((N,), jnp.float32),
            pltpu.VMEM((_D, K), jnp.int32),
            pltpu.VMEM((_D, K), jnp.int32),
            pltpu.VMEM((_D, K), jnp.float32),
            pltpu.VMEM((_D, 1, K), jnp.int32),
            pltpu.VMEM((_D, 1, K), jnp.int32),
            pltpu.VMEM((_D, K, OUT_DIM), jnp.float32),
            pltpu.VMEM((_D, K, ROWW), jnp.float32),
        ] + [pltpu.SemaphoreType.DMA] * (3 * _D),
    )
    return f(z_flat, ssrc, sdst, src, dst)


def kernel(h, edge_index, W, a):
    wcat = W.transpose(2, 0, 1).reshape(IN_DIM, HEADS * OUT_DIM)
    a1 = a[0, 0, :OUT_DIM].reshape(OUT_DIM, 1)
    a2 = a[0, 0, OUT_DIM:].reshape(OUT_DIM, 1)
    z4, s1, s2 = _tc_call(h, wcat, a1, a2)
    z_flat = z4.reshape(HEADS * N, OUT_DIM)
    ssrc = s1.reshape(HEADS, N)
    sdst = s2.reshape(HEADS, N)
    return _sc_call(z_flat, ssrc, sdst, edge_index[0], edge_index[1])


# speedup vs baseline: 60.6637x; 1.0560x over previous
import jax
import jax.numpy as jnp
from jax import lax
from jax.experimental import pallas as pl
from jax.experimental.pallas import tpu as pltpu
from jax.experimental.pallas import tpu_sc as plsc

N = 10000
E = 320000
IN_DIM = 128
OUT_DIM = 64
HEADS = 4

NC = 2
NS = 16
EPT = E // NS
K = 80
NB = EPT // K
RPT = 640
DR = 640
_D = 5


def _tc_body(h_ref, w_ref, a1_ref, a2_ref, z_ref, s1_ref, s2_ref):
    hb = h_ref[...]
    zc = lax.dot_general(hb, w_ref[...], (((1,), (0,)), ((), ())),
                         preferred_element_type=jnp.float32)
    for hd in range(HEADS):
        zh = zc[:, hd * OUT_DIM:(hd + 1) * OUT_DIM]
        z_ref[hd] = zh
        s1_ref[hd] = lax.dot_general(zh, a1_ref[...], (((1,), (0,)), ((), ())),
                                     preferred_element_type=jnp.float32)
        s2_ref[hd] = lax.dot_general(zh, a2_ref[...], (((1,), (0,)), ((), ())),
                                     preferred_element_type=jnp.float32)


_BN = 1000


def _tc_call(h, wcat, a1, a2):
    return pl.pallas_call(
        _tc_body,
        grid=(N // _BN,),
        in_specs=[
            pl.BlockSpec((_BN, IN_DIM), lambda i: (i, 0)),
            pl.BlockSpec((IN_DIM, HEADS * OUT_DIM), lambda i: (0, 0)),
            pl.BlockSpec((OUT_DIM, 1), lambda i: (0, 0)),
            pl.BlockSpec((OUT_DIM, 1), lambda i: (0, 0)),
        ],
        out_specs=[
            pl.BlockSpec((HEADS, _BN, OUT_DIM), lambda i: (0, i, 0)),
            pl.BlockSpec((HEADS, _BN, 1), lambda i: (0, i, 0)),
            pl.BlockSpec((HEADS, _BN, 1), lambda i: (0, i, 0)),
        ],
        out_shape=[
            jax.ShapeDtypeStruct((HEADS, N, OUT_DIM), jnp.float32),
            jax.ShapeDtypeStruct((HEADS, N, 1), jnp.float32),
            jax.ShapeDtypeStruct((HEADS, N, 1), jnp.float32),
        ],
    )(h, wcat, a1, a2)


def _sc_body(z_hbm, ssrc_hbm, sdst_hbm, src_hbm, dst_hbm, out_hbm,
             acc, den_sp, ssrc_v, sdst_v, srcb, dstb, wbuf, zidx, sidx,
             zg, rowbuf, den_l, iidx, dsb, zb, *sems):
    c = lax.axis_index("c")
    s = lax.axis_index("s")
    zero16 = jnp.zeros((16,), jnp.float32)
    iota16 = lax.iota(jnp.int32, 16)
    e0 = s * EPT
    sem_e = sems[0:_D]
    sem_g = sems[_D:2 * _D]
    sem_s = sems[2 * _D:3 * _D]
    row0 = s * RPT
    last = s == NS - 1

    def stage_edges(g, p):
        off = e0 + g * K
        pltpu.async_copy(src_hbm.at[pl.ds(off, K)], srcb.at[p], sem_e[p])
        pltpu.async_copy(dst_hbm.at[pl.ds(off, K)], dstb.at[p], sem_e[p])

    def wait_edges(g, p):
        off = e0 + g * K
        pltpu.make_async_copy(src_hbm.at[pl.ds(off, K)], srcb.at[p],
                              sem_e[p]).wait()
        pltpu.make_async_copy(dst_hbm.at[pl.ds(off, K)], dstb.at[p],
                              sem_e[p]).wait()

    for hl in range(2):
        head = 2 * c + hl
        pltpu.sync_copy(ssrc_hbm.at[head], ssrc_v)
        pltpu.sync_copy(sdst_hbm.at[head], sdst_v)

        def _zr(r, _):
            for j in range(OUT_DIM // 16):
                rowbuf[0, r, pl.ds(16 * j, 16)] = zero16
            return 0
        lax.fori_loop(0, K, _zr, 0)

        def _zd(r, _):
            den_l[r, pl.ds(0, 16)] = zero16
            return 0
        lax.fori_loop(0, DR, _zd, 0)

        def _zb(r, _):
            zb[r, pl.ds(0, 16)] = zero16
            return 0
        lax.fori_loop(0, 40, _zb, 0)
        pltpu.sync_copy(zb, den_sp.at[pl.ds(s * 40, 40)])

        for kk in range(8):
            def _zc(kk=kk):
                pltpu.sync_copy(rowbuf.at[0],
                                acc.at[pl.ds(row0 + kk * K, K)])
            if kk < 5:
                _zc()
            else:
                pl.when(jnp.logical_not(last))(_zc)
        plsc.subcore_barrier()

        def _mk_w(p, head=head):
            for grp in range(K // 16):
                sv = srcb[p, pl.ds(grp * 16, 16)]
                dv = dstb[p, pl.ds(grp * 16, 16)]
                s1 = plsc.load_gather(ssrc_v, [sv])
                s2 = plsc.load_gather(sdst_v, [dv])
                e = s1 + s2
                e = jnp.where(e > 0.0, e, e * jnp.float32(0.01))
                wbuf[p, pl.ds(grp * 16, 16)] = jnp.exp(e)
                zidx[p, 0, pl.ds(grp * 16, 16)] = sv + head * N
            pltpu.async_copy(z_hbm.at[zidx.at[p, 0]], zg.at[p], sem_g[p])

        def _wait_gather(p):
            pltpu.make_async_copy(z_hbm.at[zidx.at[p, 0]], zg.at[p],
                                  sem_g[p]).wait()

        def _wait_scatter(p):
            pltpu.make_async_copy(rowbuf.at[p], acc.at[sidx.at[p, 0]],
                                  sem_s[p]).wait()

        def _mul_scatter(p, b=None):
            def mb(q, _):
                wv = wbuf[p, pl.ds(16 * q, 16)]
                dv = dstb[p, pl.ds(16 * q, 16)]
                plsc.addupdate_scatter(
                    den_l, [jnp.right_shift(dv, 4),
                            jnp.bitwise_and(dv, 15)], wv)
                for e2 in range(4):
                    blk = []
                    for e_i in range(4):
                        r = 16 * q + 4 * e2 + e_i
                        w_s = wv[4 * e2 + e_i]
                        vals = [zg[p, r, pl.ds(16 * j, 16)] * w_s
                                for j in range(OUT_DIM // 16)]
                        blk.append((r, vals))
                    for r, vals in blk:
                        for j in range(OUT_DIM // 16):
                            rowbuf[p, r, pl.ds(16 * j, 16)] = vals[j]
                return 0
            lax.fori_loop(0, K // 16, mb, 0)
            for grp in range(K // 16):
                sidx[p, 0, pl.ds(grp * 16, 16)] = (
                    dstb[p, pl.ds(grp * 16, 16)])
            if b is not None:
                pl.when(b > 0)(
                    lambda: _wait_scatter((p + _D - 1) % _D))
            pltpu.async_copy(rowbuf.at[p], acc.at[sidx.at[p, 0]], sem_s[p],
                             add=True)

        for g in range(_D - 1):
            stage_edges(g, g)
        for g in range(2):
            wait_edges(g, g)
            _mk_w(g)

        def _lb(t, _):
            for i in range(_D):
                b = _D * t + i
                sS = (i + 4) % _D
                sX = (i + 2) % _D
                pl.when(b + 4 < NB)(
                    lambda b=b, sS=sS: stage_edges(b + 4, sS))

                def _x(b=b, sX=sX):
                    wait_edges(b + 2, sX)
                    _mk_w(sX)
                pl.when(b + 2 < NB)(_x)
                _wait_gather(i)
                _mul_scatter(i, b)
            return 0
        lax.fori_loop(0, NB // _D, _lb, 0)
        _wait_scatter((NB - 1) % _D)

        for ch in range(DR // 128):
            for m in range(8):
                iidx[pl.ds(16 * m, 16)] = ch * 128 + 16 * m + iota16
            pltpu.sync_copy(den_l.at[pl.ds(ch * 128, 128)],
                            den_sp.at[iidx], add=True)

        plsc.subcore_barrier()

        col = head * OUT_DIM
        for kk in range(8):
            def _ep(kk=kk):
                r0 = row0 + kk * K
                pltpu.sync_copy(acc.at[pl.ds(r0, K)], rowbuf.at[0])
                pltpu.sync_copy(den_sp.at[pl.ds(s * 40 + kk * 5, 5)], dsb)

                def _db(r16, _):
                    den16 = dsb[r16, pl.ds(0, 16)]
                    inv16 = jnp.where(den16 > 0.0, 1.0 / den16, 0.0)
                    for e2 in range(4):
                        blk = []
                        for e_i in range(4):
                            r = 16 * r16 + 4 * e2 + e_i
                            inv = inv16[4 * e2 + e_i]
                            vals = [rowbuf[0, r, pl.ds(16 * j, 16)] * inv
                                    for j in range(OUT_DIM // 16)]
                            blk.append((r, vals))
                        for r, vals in blk:
                            for j in range(OUT_DIM // 16):
                                rowbuf[0, r, pl.ds(16 * j, 16)] = vals[j]
                    return 0
                lax.fori_loop(0, K // 16, _db, 0)
                pltpu.sync_copy(rowbuf.at[0],
                                out_hbm.at[pl.ds(r0, K),
                                           pl.ds(col, OUT_DIM)])
            if kk < 5:
                _ep()
            else:
                pl.when(jnp.logical_not(last))(_ep)
        plsc.subcore_barrier()


def _sc_call(z_flat, ssrc, sdst, src, dst):
    mesh = plsc.VectorSubcoreMesh(core_axis_name="c", subcore_axis_name="s",
                                  num_cores=NC, num_subcores=NS)
    f = pl.kernel(
        _sc_body,
        out_type=jax.ShapeDtypeStruct((N, HEADS * OUT_DIM), jnp.float32),
        mesh=mesh,
        compiler_params=pltpu.CompilerParams(use_tc_tiling_on_sc=False,
                                             needs_layout_passes=False),
        scratch_types=[
            pltpu.VMEM_SHARED((N, OUT_DIM), jnp.float32),
            pltpu.VMEM_SHARED((DR, 16), jnp.float32),
            pltpu.VMEM((N,), jnp.float32),
            pltpu.VMEM((N,), jnp.float32),
            pltpu.VMEM((_D, K), jnp.int32),
            pltpu.VMEM((_D, K), jnp.int32),
            pltpu.VMEM((_D, K), jnp.float32),
            pltpu.VMEM((_D, 1, K), jnp.int32),
            pltpu.VMEM((_D, 1, K), jnp.int32),
            pltpu.VMEM((_D, K, OUT_DIM), jnp.float32),
            pltpu.VMEM((_D, K, OUT_DIM), jnp.float32),
            pltpu.VMEM((DR, 16), jnp.float32),
            pltpu.VMEM((128,), jnp.int32),
            pltpu.VMEM((5, 16), jnp.float32),
            pltpu.VMEM((40, 16), jnp.float32),
        ] + [pltpu.SemaphoreType.DMA] * (3 * _D),
    )
    return f(z_flat, ssrc, sdst, src, dst)


def kernel(h, edge_index, W, a):
    wcat = W.transpose(2, 0, 1).reshape(IN_DIM, HEADS * OUT_DIM)
    a1 = a[0, 0, :OUT_DIM].reshape(OUT_DIM, 1)
    a2 = a[0, 0, OUT_DIM:].reshape(OUT_DIM, 1)
    z4, s1, s2 = _tc_call(h, wcat, a1, a2)
    z_flat = z4.reshape(HEADS * N, OUT_DIM)
    ssrc = s1.reshape(HEADS, N)
    sdst = s2.reshape(HEADS, N)
    return _sc_call(z_flat, ssrc, sdst, edge_index[0], edge_index[1])
